# Initial kernel scaffold; baseline (speedup 1.0000x reference)
import jax
import jax.numpy as jnp
from jax.experimental import pallas as pl


def kernel(x, edge_index, bn_gamma, bn_beta, enc_W1, enc_b1, enc_W2, enc_b2, enc_W3, enc_b3, dec_W1, dec_b1, dec_W2, dec_b2, dec_W3, dec_b3):
    raise NotImplementedError("write your pallas kernel here")


# speedup vs baseline: 3.7427x; 3.7427x over previous
import functools

import jax
import jax.numpy as jnp
from jax import lax
from jax.experimental import pallas as pl
from jax.experimental.pallas import tpu as pltpu
from jax.experimental.pallas import tpu_sc as plsc

N = 100000
E = 1600000
LANES = 128

NC = 2
NS = 16
NW = NC * NS

CHUNK = 128
KCH = 8
OUTER = 49
PER_TILE = OUTER * KCH
EC_P = PER_TILE * NW
EP = EC_P * CHUNK
GANG = KCH * CHUNK

BE = 8192
BN_BLK = 5000
INIT_ROWS = 6256
INIT_ROWS_LAST = N - 15 * INIT_ROWS


def _bn_body(x_ref, g_ref, b_ref, o_ref):
    xv = x_ref[...]
    n = jnp.float32(N)
    sums = jnp.sum(xv, axis=0, keepdims=True)
    sumsq = jnp.sum(xv * xv, axis=0, keepdims=True)
    r = lax.broadcasted_iota(jnp.int32, (LANES, 4), 0)
    c = lax.broadcasted_iota(jnp.int32, (LANES, 4), 1)
    fold = (r % 4 == c).astype(jnp.float32)
    s4 = jnp.dot(sums, fold, preferred_element_type=jnp.float32)
    q4 = jnp.dot(sumsq, fold, preferred_element_type=jnp.float32)
    mean = s4 / n
    var = q4 / n - mean * mean
    scale = g_ref[...] / jnp.sqrt(var + 1e-5)
    shift = b_ref[...] - mean * scale
    r2 = lax.broadcasted_iota(jnp.int32, (4, LANES), 0)
    c2 = lax.broadcasted_iota(jnp.int32, (4, LANES), 1)
    unfold = (c2 % 4 == r2).astype(jnp.float32)
    s128 = jnp.dot(scale, unfold, preferred_element_type=jnp.float32)
    h128 = jnp.dot(shift, unfold, preferred_element_type=jnp.float32)
    o_ref[...] = xv * s128 + h128


def _bn(x_packed, gamma, beta):
    return pl.pallas_call(
        _bn_body,
        out_shape=jax.ShapeDtypeStruct(x_packed.shape, jnp.float32),
    )(x_packed, gamma, beta)


def _sc_gather_body(table, dst2, src2, hd_out, hs_out,
                    idxd_v, idxs_v, rowd_v, rows_v, sem):
    cid = lax.axis_index("c")
    sid = lax.axis_index("s")
    wid = cid * NS + sid
    start_chunk = wid * PER_TILE

    def gang(chunk0, idx2, idx_v, row_v, out):
        pltpu.sync_copy(idx2.at[pl.ds(chunk0, KCH)], idx_v)
        descs = []
        for j in range(KCH):
            descs.append(
                pltpu.async_copy(table.at[idx_v.at[j]],
                                 row_v.at[pl.ds(j * CHUNK, CHUNK)], sem))
        for d in descs:
            d.wait()
        pltpu.sync_copy(row_v, out.at[pl.ds(chunk0 * CHUNK, GANG)])

    def body(i, carry):
        chunk0 = start_chunk + i * KCH
        gang(chunk0, dst2, idxd_v, rowd_v, hd_out)
        gang(chunk0, src2, idxs_v, rows_v, hs_out)
        return carry

    lax.fori_loop(0, OUTER, body, 0)


def _sc_gather(table, dst2, src2):
    mesh = plsc.VectorSubcoreMesh(core_axis_name="c", subcore_axis_name="s")
    f = pl.kernel(
        _sc_gather_body,
        out_type=(jax.ShapeDtypeStruct((EP, 4), jnp.float32),
                  jax.ShapeDtypeStruct((EP, 4), jnp.float32)),
        mesh=mesh,
        scratch_types=[
            pltpu.VMEM((KCH, CHUNK), jnp.int32),
            pltpu.VMEM((KCH, CHUNK), jnp.int32),
            pltpu.VMEM((GANG, 4), jnp.float32),
            pltpu.VMEM((GANG, 4), jnp.float32),
            pltpu.SemaphoreType.DMA,
        ],
        compiler_params=pltpu.CompilerParams(use_tc_tiling_on_sc=False),
    )
    return f(table, dst2, src2)


def _sc_scatter_body(vals, dst2, zeros_hbm, out, *rest):
    idx_bufs = rest[:KCH]
    val_bufs = rest[KCH:2 * KCH]
    acc, sem, sem2 = rest[2 * KCH:]
    cid = lax.axis_index("c")
    sid = lax.axis_index("s")
    wid = cid * NS + sid
    start_chunk = wid * PER_TILE

    r0 = sid * INIT_ROWS

    @pl.when(sid < NS - 1)
    def _():
        pltpu.sync_copy(zeros_hbm.at[pl.ds(r0, INIT_ROWS)],
                        acc.at[pl.ds(r0, INIT_ROWS)])

    @pl.when(sid == NS - 1)
    def _():
        pltpu.sync_copy(zeros_hbm.at[pl.ds(r0, INIT_ROWS_LAST)],
                        acc.at[pl.ds(r0, INIT_ROWS_LAST)])

    plsc.subcore_barrier()

    def gang(chunk0):
        loads = []
        for j in range(KCH):
            loads.append(pltpu.async_copy(dst2.at[chunk0 + j],
                                          idx_bufs[j], sem2))
            loads.append(pltpu.async_copy(
                vals.at[pl.ds((chunk0 + j) * CHUNK, CHUNK)],
                val_bufs[j], sem2))
        for d in loads:
            d.wait()
        adds = []
        for j in range(KCH):
            adds.append(pltpu.async_copy(val_bufs[j], acc.at[idx_bufs[j]],
                                         sem, add=True))
        for d in adds:
            d.wait()

    def body(i, carry):
        gang(start_chunk + i * KCH)
        return carry

    lax.fori_loop(0, OUTER, body, 0)

    plsc.subcore_barrier()

    @pl.when(sid < NS - 1)
    def _():
        pltpu.sync_copy(acc.at[pl.ds(r0, INIT_ROWS)],
                        out.at[cid, pl.ds(r0, INIT_ROWS)])

    @pl.when(sid == NS - 1)
    def _():
        pltpu.sync_copy(acc.at[pl.ds(r0, INIT_ROWS_LAST)],
                        out.at[cid, pl.ds(r0, INIT_ROWS_LAST)])


def _sc_scatter(vals, dst2, zeros_hbm):
    mesh = plsc.VectorSubcoreMesh(core_axis_name="c", subcore_axis_name="s")
    f = pl.kernel(
        _sc_scatter_body,
        out_type=jax.ShapeDtypeStruct((NC, N, 4), jnp.float32),
        mesh=mesh,
        scratch_types=(
            [pltpu.VMEM((CHUNK,), jnp.int32) for _ in range(KCH)]
            + [pltpu.VMEM((CHUNK, 4), jnp.float32) for _ in range(KCH)]
            + [pltpu.VMEM_SHARED((N, 4), jnp.float32),
               pltpu.SemaphoreType.DMA,
               pltpu.SemaphoreType.DMA]
        ),
        compiler_params=pltpu.CompilerParams(use_tc_tiling_on_sc=False),
    )
    return f(vals, dst2, zeros_hbm)


def _mlp_body(hd_ref, hs_ref, wa_ref, wb_ref, b1_ref, w2_ref, b2_ref,
              w3_ref, b3_ref, o_ref, *, relu3):
    hi = hd_ref[...]
    hj = hs_ref[...]
    z = (jnp.dot(hi, wa_ref[...], preferred_element_type=jnp.float32)
         + jnp.dot(hj, wb_ref[...], preferred_element_type=jnp.float32)
         + b1_ref[...])
    z = jnp.maximum(z, 0.0)
    z = jnp.dot(z, w2_ref[...], preferred_element_type=jnp.float32) + b2_ref[...]
    z = jnp.maximum(z, 0.0)
    z = jnp.dot(z, w3_ref[...], preferred_element_type=jnp.float32) + b3_ref[...]
    if relu3:
        z = jnp.maximum(z, 0.0)
    row = pl.program_id(0) * BE + lax.broadcasted_iota(jnp.int32, (BE, 4), 0)
    o_ref[...] = jnp.where(row < E, z, 0.0)


def _mlp(hd, hs, wa, wb, b1, w2, b2, w3, b3, relu3):
    grid = (EP // BE,)
    edge_spec = pl.BlockSpec((BE, 4), lambda i: (i, 0))

    def wspec(shape):
        return pl.BlockSpec(shape, lambda i: (0, 0))

    return pl.pallas_call(
        functools.partial(_mlp_body, relu3=relu3),
        grid=grid,
        in_specs=[edge_spec, edge_spec,
                  wspec((4, 32)), wspec((4, 32)), wspec((1, 32)),
                  wspec((32, 32)), wspec((1, 32)),
                  wspec((32, 4)), wspec((1, 4))],
        out_specs=edge_spec,
        out_shape=jax.ShapeDtypeStruct((EP, 4), jnp.float32),
    )(hd, hs, wa, wb, b1, w2, b2, w3, b3)


def _fin1_body(p_ref, h_ref, inv_ref):
    s = p_ref[0] + p_ref[1]
    lane = lax.broadcasted_iota(jnp.int32, (BN_BLK, 4), 1)
    cnt = jnp.sum(jnp.where(lane == 2, s, 0.0), axis=1, keepdims=True)
    inv = 1.0 / jnp.maximum(cnt, 1.0)
    h_ref[...] = jnp.where(lane < 2, s * inv, 0.0)
    inv_ref[...] = jnp.broadcast_to(inv, (BN_BLK, 4))


def _fin1(parts):
    grid = (N // BN_BLK,)
    return pl.pallas_call(
        _fin1_body,
        grid=grid,
        in_specs=[pl.BlockSpec((NC, BN_BLK, 4), lambda i: (0, i, 0))],
        out_specs=[pl.BlockSpec((BN_BLK, 4), lambda i: (i, 0))] * 2,
        out_shape=[jax.ShapeDtypeStruct((N, 4), jnp.float32)] * 2,
    )(parts)


def _fin2_body(p_ref, inv_ref, o_ref):
    o_ref[...] = (p_ref[0] + p_ref[1]) * inv_ref[...]


def _fin2(parts, inv4):
    grid = (N // BN_BLK,)
    return pl.pallas_call(
        _fin2_body,
        grid=grid,
        in_specs=[pl.BlockSpec((NC, BN_BLK, 4), lambda i: (0, i, 0)),
                  pl.BlockSpec((BN_BLK, 4), lambda i: (i, 0))],
        out_specs=pl.BlockSpec((BN_BLK, 4), lambda i: (i, 0)),
        out_shape=jax.ShapeDtypeStruct((N, 4), jnp.float32),
    )(parts, inv4)


def kernel(x, edge_index, bn_gamma, bn_beta,
           enc_W1, enc_b1, enc_W2, enc_b2, enc_W3, enc_b3,
           dec_W1, dec_b1, dec_W2, dec_b2, dec_W3, dec_b3):
    f32 = jnp.float32
    pad = jnp.zeros((EP - E,), jnp.int32)
    src2 = jnp.concatenate([edge_index[0], pad]).reshape(EC_P, CHUNK)
    dst2 = jnp.concatenate([edge_index[1], pad]).reshape(EC_P, CHUNK)
    x_packed = x.reshape(N * 4 // LANES, LANES)
    zeros_n4 = jnp.zeros((N, 4), f32)

    e_wa = enc_W1[:4] - enc_W1[4:]
    e_wb = enc_W1[4:]
    e_b1 = enc_b1.reshape(1, 32)
    e_b2 = enc_b2.reshape(1, 32)
    e_w3 = jnp.concatenate([enc_W3, jnp.zeros((32, 2), f32)], axis=1)
    e_b3 = jnp.concatenate([enc_b3, jnp.array([1.0, 0.0], f32)]).reshape(1, 4)

    d_wa = jnp.concatenate([dec_W1[:2] - dec_W1[2:], jnp.zeros((2, 32), f32)])
    d_wb = jnp.concatenate([dec_W1[2:], jnp.zeros((2, 32), f32)])
    d_b1 = dec_b1.reshape(1, 32)
    d_b2 = dec_b2.reshape(1, 32)
    d_b3 = dec_b3.reshape(1, 4)

    xn = _bn(x_packed, bn_gamma.reshape(1, 4), bn_beta.reshape(1, 4))
    xn4 = xn.reshape(N, 4)

    def xla_scatter(m):
        full = jax.ops.segment_sum(m[:E], edge_index[1], num_segments=N)
        return jnp.stack([full, jnp.zeros((N, 4), f32)])

    hd, hs = _sc_gather(xn4, dst2, src2)
    m1 = _mlp(hd, hs, e_wa, e_wb, e_b1, enc_W2, e_b2, e_w3, e_b3, relu3=True)
    parts1 = _sc_scatter(m1, dst2, zeros_n4)
    h4, inv4 = _fin1(parts1)

    hd2, hs2 = _sc_gather(h4, dst2, src2)
    m2 = _mlp(hd2, hs2, d_wa, d_wb, d_b1, dec_W2, d_b2, dec_W3, d_b3,
              relu3=False)
    parts2 = _sc_scatter(m2, dst2, zeros_n4)
    return _fin2(parts2, inv4)
